# baseline (speedup 1.0000x reference)
import functools

import jax
import jax.numpy as jnp
from jax import lax
from jax.experimental import pallas as pl
from jax.experimental.pallas import tpu as pltpu
from jax.experimental.pallas import tpu_sc as plsc


def kernel(tokens, W_pos):
    batch, seq_len = tokens.shape
    d_model = W_pos.shape[1]

    info = plsc.get_sparse_core_info()
    NC, NS = info.num_cores, info.num_subcores
    NW = NC * NS
    rows_per_w = seq_len // NW
    C = 32
    n_chunks = rows_per_w // C

    mesh = plsc.VectorSubcoreMesh(core_axis_name="c", subcore_axis_name="s")

    @functools.partial(
        pl.kernel,
        mesh=mesh,
        out_type=jax.ShapeDtypeStruct((batch, seq_len, d_model), W_pos.dtype),
        scratch_types=[pltpu.VMEM((C, d_model), jnp.float32)],
    )
    def sc_broadcast(w_hbm, out_hbm, buf):
        wid = lax.axis_index("s") * NC + lax.axis_index("c")
        base0 = wid * rows_per_w
        for c in range(n_chunks):
            base = base0 + c * C
            pltpu.sync_copy(w_hbm.at[pl.ds(base, C)], buf)
            for b in range(batch):
                pltpu.sync_copy(buf, out_hbm.at[b, pl.ds(base, C)])

    return sc_broadcast(W_pos)


# speedup vs baseline: 1.0191x; 1.0191x over previous
import functools

import jax
import jax.numpy as jnp
from jax import lax
from jax.experimental import pallas as pl
from jax.experimental.pallas import tpu as pltpu
from jax.experimental.pallas import tpu_sc as plsc


def kernel(tokens, W_pos):
    batch, seq_len = tokens.shape
    d_model = W_pos.shape[1]

    info = plsc.get_sparse_core_info()
    NC, NS = info.num_cores, info.num_subcores
    NW = NC * NS
    rows_per_w = seq_len // NW
    C = 32
    n_chunks = rows_per_w // C

    mesh = plsc.VectorSubcoreMesh(core_axis_name="c", subcore_axis_name="s")

    @functools.partial(
        pl.kernel,
        mesh=mesh,
        out_type=jax.ShapeDtypeStruct((batch, seq_len, d_model), W_pos.dtype),
        scratch_types=[
            pltpu.VMEM((C, d_model), jnp.float32),
            pltpu.VMEM_SHARED((NS, C, d_model), jnp.float32),
            pltpu.SemaphoreType.DMA,
            pltpu.SemaphoreType.DMA,
        ],
    )
    def sc_broadcast(w_hbm, out_hbm, buf, sbuf, rsem, wsem):
        cid = lax.axis_index("c")
        sid = lax.axis_index("s")
        wid = sid * NC + cid
        base0 = wid * rows_per_w
        baseB = base0 + (n_chunks - 1) * C
        hb_read = pltpu.async_copy(w_hbm.at[pl.ds(baseB, C)], sbuf.at[sid], rsem)
        hb_writes = None
        for c in range(n_chunks - 1):
            base = base0 + c * C
            pltpu.sync_copy(w_hbm.at[pl.ds(base, C)], buf)
            for b in range(batch):
                pltpu.sync_copy(buf, out_hbm.at[b, pl.ds(base, C)])
            if c == 0:
                hb_read.wait()
                hb_writes = [
                    pltpu.async_copy(
                        sbuf.at[sid], out_hbm.at[b, pl.ds(baseB, C)], wsem)
                    for b in range(batch)
                ]
        for h in hb_writes:
            h.wait()

    return sc_broadcast(W_pos)
